# baseline (speedup 1.0000x reference)
import functools

import jax
import jax.numpy as jnp
from jax import lax
from jax.experimental import pallas as pl
from jax.experimental.pallas import tpu as pltpu
from jax.experimental.pallas import tpu_sc as plsc

N = 10000
E = 320000
D = 128
G = 8

NC = 2
NS = 16
NW = NC * NS
CH = 128
NCHUNK = 80
PADE = NW * NCHUNK * CH
NOUTER = NCHUNK // 4
RPT = 624
TAIL = N - NS * RPT
ZROWS = 48

_mesh = plsc.VectorSubcoreMesh(core_axis_name="c", subcore_axis_name="s")


@functools.partial(
    pl.kernel,
    out_type=jax.ShapeDtypeStruct((NC, N, D), jnp.float32),
    mesh=_mesh,
    scratch_types=[
        [pltpu.VMEM((CH,), jnp.int32) for _ in range(4)],
        [pltpu.VMEM((CH,), jnp.int32) for _ in range(4)],
        [pltpu.VMEM((CH, D), jnp.float32) for _ in range(2)],
        pltpu.VMEM((ZROWS, D), jnp.float32),
        pltpu.VMEM_SHARED((N + 1024, D), jnp.float32),
        pltpu.SemaphoreType.DMA,
        pltpu.SemaphoreType.DMA((2,)),
        pltpu.SemaphoreType.DMA((4,)),
        pltpu.SemaphoreType.DMA((4,)),
    ],
)
def _sc_agg(h_hbm, src_hbm, dst_hbm, out_hbm,
            si, di, rows, zbuf, acc_sh, sem_g, sem_s, sem_si, sem_di):
    c = lax.axis_index("c")
    s = lax.axis_index("s")
    wid = c * NS + s

    zv = jnp.zeros((16,), jnp.float32)

    def _zrow(i, _):
        def _zcol(j, _):
            zbuf[i, pl.ds(j * 16, 16)] = zv
            return 0
        return lax.fori_loop(0, D // 16, _zcol, 0)

    lax.fori_loop(0, ZROWS, _zrow, 0)

    def _zcp(k, _):
        pltpu.sync_copy(zbuf, acc_sh.at[pl.ds(s * RPT + k * ZROWS, ZROWS)])
        return 0

    lax.fori_loop(0, RPT // ZROWS, _zcp, 0)

    @pl.when(s == NS - 1)
    def _ztail():
        pltpu.sync_copy(zbuf.at[pl.ds(0, TAIL)], acc_sh.at[pl.ds(NS * RPT, TAIL)])

    plsc.subcore_barrier()

    base = wid * NCHUNK * CH

    def _issue_idx(off, j):
        pltpu.async_copy(src_hbm.at[pl.ds(off, CH)], si[j], sem_si.at[j])
        pltpu.async_copy(dst_hbm.at[pl.ds(off, CH)], di[j], sem_di.at[j])

    _issue_idx(base, 0)
    _issue_idx(base + CH, 1)

    def _outer(g, _):
        for u in range(4):
            b = u % 2
            off = base + (4 * g + u) * CH

            def _drain(b=b, u=u):
                pltpu.make_async_copy(
                    rows[b], acc_sh.at[di[(u + 2) % 4]], sem_s.at[b]).wait()

            if u < 2:
                pl.when(g > 0)(_drain)
            else:
                _drain()

            pltpu.make_async_copy(
                src_hbm.at[pl.ds(off, CH)], si[u], sem_si.at[u]).wait()
            pltpu.make_async_copy(
                dst_hbm.at[pl.ds(off, CH)], di[u], sem_di.at[u]).wait()

            pltpu.async_copy(h_hbm.at[si[u]], rows[b], sem_g).wait()
            pltpu.async_copy(rows[b], acc_sh.at[di[u]], sem_s.at[b], add=True)

            def _prefetch(off=off, u=u):
                _issue_idx(off + 2 * CH, (u + 2) % 4)

            if u < 2:
                _prefetch()
            else:
                pl.when(g < NOUTER - 1)(_prefetch)
        return 0

    lax.fori_loop(0, NOUTER, _outer, 0)

    pltpu.make_async_copy(rows[0], acc_sh.at[di[2]], sem_s.at[0]).wait()
    pltpu.make_async_copy(rows[1], acc_sh.at[di[3]], sem_s.at[1]).wait()

    plsc.subcore_barrier()

    pltpu.sync_copy(acc_sh.at[pl.ds(s * RPT, RPT)],
                    out_hbm.at[c, pl.ds(s * RPT, RPT)])

    @pl.when(s == NS - 1)
    def _wtail():
        pltpu.sync_copy(acc_sh.at[pl.ds(NS * RPT, TAIL)],
                        out_hbm.at[c, pl.ds(NS * RPT, TAIL)])


def _mlp_pool_body(emit_h, h_ref, agg_ref, batch_ref,
                   W1_ref, b1_ref, g_ref, be_ref, W2_ref, b2_ref, *outs):
    z = h_ref[...] + agg_ref[0] + agg_ref[1]
    z = jnp.dot(z, W1_ref[...], preferred_element_type=jnp.float32) + b1_ref[...]
    mean = jnp.mean(z, axis=0, keepdims=True)
    var = jnp.mean(z * z, axis=0, keepdims=True) - mean * mean
    z = (z - mean) * (g_ref[...] * lax.rsqrt(var + 1e-5)) + be_ref[...]
    z = jnp.maximum(z, 0.0)
    z = jnp.dot(z, W2_ref[...], preferred_element_type=jnp.float32) + b2_ref[...]
    h = jnp.maximum(z, 0.0)
    onehot = (batch_ref[...] ==
              lax.broadcasted_iota(jnp.int32, (G, N), 0)).astype(jnp.float32)
    pool = jnp.dot(onehot, h, preferred_element_type=jnp.float32)
    if emit_h:
        outs[0][...] = h
        outs[1][...] = pool
    else:
        outs[0][...] = pool


def _tc_layer(h, agg2, batch2d, W1, b1, gamma, beta, W2, b2, emit_h):
    if emit_h:
        out_shape = (jax.ShapeDtypeStruct((N, D), jnp.float32),
                     jax.ShapeDtypeStruct((G, D), jnp.float32))
    else:
        out_shape = (jax.ShapeDtypeStruct((G, D), jnp.float32),)
    return pl.pallas_call(
        functools.partial(_mlp_pool_body, emit_h),
        out_shape=out_shape,
    )(h, agg2, batch2d,
      W1, b1.reshape(1, D), gamma.reshape(1, D), beta.reshape(1, D),
      W2, b2.reshape(1, D))


def kernel(x, edge_index, batch_node_id,
           W1_0, b1_0, gamma_0, beta_0, W2_0, b2_0,
           W1_1, b1_1, gamma_1, beta_1, W2_1, b2_1):
    pad = PADE - E
    src = jnp.concatenate([edge_index[0], jnp.zeros((pad,), jnp.int32)])
    dst = jnp.concatenate(
        [edge_index[1], N + (jnp.arange(pad, dtype=jnp.int32) % 1024)])
    batch2d = batch_node_id.reshape(1, N)

    agg_x = _sc_agg(x, src, dst)
    h1, pool1 = _tc_layer(x, agg_x, batch2d,
                          W1_0, b1_0, gamma_0, beta_0, W2_0, b2_0, True)
    agg_h1 = _sc_agg(h1, src, dst)
    (pool2,) = _tc_layer(h1, agg_h1, batch2d,
                         W1_1, b1_1, gamma_1, beta_1, W2_1, b2_1, False)
    return jnp.concatenate([pool1, pool2], axis=1)


# speedup vs baseline: 3.5077x; 3.5077x over previous
import functools

import jax
import jax.numpy as jnp
from jax import lax
from jax.experimental import pallas as pl
from jax.experimental.pallas import tpu as pltpu
from jax.experimental.pallas import tpu_sc as plsc

N = 10000
E = 320000
D = 128
G = 8

NC = 2
NS = 16
NW = NC * NS
CH = 128
NCHUNK = 80
PADE = NW * NCHUNK * CH
NOUTER = NCHUNK // 4
RPT = 624
TAIL = N - NS * RPT
ZROWS = 48

_mesh = plsc.VectorSubcoreMesh(core_axis_name="c", subcore_axis_name="s")


@functools.partial(
    pl.kernel,
    out_type=jax.ShapeDtypeStruct((NC, N, D), jnp.float32),
    mesh=_mesh,
    scratch_types=[
        [pltpu.VMEM((CH,), jnp.int32) for _ in range(4)],
        [pltpu.VMEM((CH,), jnp.int32) for _ in range(4)],
        [pltpu.VMEM((CH, D), jnp.float32) for _ in range(2)],
        pltpu.VMEM((ZROWS, D), jnp.float32),
        pltpu.VMEM_SHARED((N + 1024, D), jnp.float32),
        pltpu.SemaphoreType.DMA,
        pltpu.SemaphoreType.DMA((2,)),
        pltpu.SemaphoreType.DMA((4,)),
        pltpu.SemaphoreType.DMA((4,)),
    ],
)
def _sc_agg(h_hbm, src_hbm, dst_hbm, out_hbm,
            si, di, rows, zbuf, acc_sh, sem_g, sem_s, sem_si, sem_di):
    c = lax.axis_index("c")
    s = lax.axis_index("s")
    wid = c * NS + s

    zv = jnp.zeros((16,), jnp.float32)

    def _zrow(i, _):
        def _zcol(j, _):
            zbuf[i, pl.ds(j * 16, 16)] = zv
            return 0
        return lax.fori_loop(0, D // 16, _zcol, 0)

    lax.fori_loop(0, ZROWS, _zrow, 0)

    def _zcp(k, _):
        pltpu.sync_copy(zbuf, acc_sh.at[pl.ds(s * RPT + k * ZROWS, ZROWS)])
        return 0

    lax.fori_loop(0, RPT // ZROWS, _zcp, 0)

    @pl.when(s == NS - 1)
    def _ztail():
        pltpu.sync_copy(zbuf.at[pl.ds(0, TAIL)], acc_sh.at[pl.ds(NS * RPT, TAIL)])

    plsc.subcore_barrier()

    base = wid * NCHUNK * CH

    def _issue_idx(off, j):
        pltpu.async_copy(src_hbm.at[pl.ds(off, CH)], si[j], sem_si.at[j])
        pltpu.async_copy(dst_hbm.at[pl.ds(off, CH)], di[j], sem_di.at[j])

    _issue_idx(base, 0)
    _issue_idx(base + CH, 1)

    def _outer(g, _):
        for u in range(4):
            b = u % 2
            off = base + (4 * g + u) * CH

            def _drain(b=b, u=u):
                pltpu.make_async_copy(
                    rows[b], acc_sh.at[di[(u + 2) % 4]], sem_s.at[b]).wait()

            if u < 2:
                pl.when(g > 0)(_drain)
            else:
                _drain()

            pltpu.make_async_copy(
                src_hbm.at[pl.ds(off, CH)], si[u], sem_si.at[u]).wait()
            pltpu.make_async_copy(
                dst_hbm.at[pl.ds(off, CH)], di[u], sem_di.at[u]).wait()

            pltpu.async_copy(h_hbm.at[si[u]], rows[b], sem_g).wait()
            pltpu.async_copy(rows[b], acc_sh.at[di[u]], sem_s.at[b], add=True)

            def _prefetch(off=off, u=u):
                _issue_idx(off + 2 * CH, (u + 2) % 4)

            if u < 2:
                _prefetch()
            else:
                pl.when(g < NOUTER - 1)(_prefetch)
        return 0

    lax.fori_loop(0, NOUTER, _outer, 0)

    pltpu.make_async_copy(rows[0], acc_sh.at[di[2]], sem_s.at[0]).wait()
    pltpu.make_async_copy(rows[1], acc_sh.at[di[3]], sem_s.at[1]).wait()

    plsc.subcore_barrier()

    pltpu.sync_copy(acc_sh.at[pl.ds(s * RPT, RPT)],
                    out_hbm.at[c, pl.ds(s * RPT, RPT)])

    @pl.when(s == NS - 1)
    def _wtail():
        pltpu.sync_copy(acc_sh.at[pl.ds(NS * RPT, TAIL)],
                        out_hbm.at[c, pl.ds(NS * RPT, TAIL)])


def _mlp_pool_body(emit_h, h_ref, agg_ref, batch_ref,
                   W1_ref, b1_ref, g_ref, be_ref, W2_ref, b2_ref, *outs):
    z = h_ref[...] + agg_ref[0] + agg_ref[1]
    z = jnp.dot(z, W1_ref[...], preferred_element_type=jnp.float32) + b1_ref[...]
    mean = jnp.mean(z, axis=0, keepdims=True)
    var = jnp.mean(z * z, axis=0, keepdims=True) - mean * mean
    z = (z - mean) * (g_ref[...] * lax.rsqrt(var + 1e-5)) + be_ref[...]
    z = jnp.maximum(z, 0.0)
    z = jnp.dot(z, W2_ref[...], preferred_element_type=jnp.float32) + b2_ref[...]
    h = jnp.maximum(z, 0.0)
    onehot = (batch_ref[...] ==
              lax.broadcasted_iota(jnp.int32, (G, N), 0)).astype(jnp.float32)
    pool = jnp.dot(onehot, h, preferred_element_type=jnp.float32)
    if emit_h:
        outs[0][...] = h
        outs[1][...] = pool
    else:
        outs[0][...] = pool


def _tc_layer(h, agg2, batch2d, W1, b1, gamma, beta, W2, b2, emit_h):
    if emit_h:
        out_shape = (jax.ShapeDtypeStruct((N, D), jnp.float32),
                     jax.ShapeDtypeStruct((G, D), jnp.float32))
    else:
        out_shape = (jax.ShapeDtypeStruct((G, D), jnp.float32),)
    return pl.pallas_call(
        functools.partial(_mlp_pool_body, emit_h),
        out_shape=out_shape,
    )(h, agg2, batch2d,
      W1, b1.reshape(1, D), gamma.reshape(1, D), beta.reshape(1, D),
      W2, b2.reshape(1, D))


def kernel(x, edge_index, batch_node_id,
           W1_0, b1_0, gamma_0, beta_0, W2_0, b2_0,
           W1_1, b1_1, gamma_1, beta_1, W2_1, b2_1):
    pad = PADE - E
    src = jnp.concatenate(
        [edge_index[0], jnp.arange(pad, dtype=jnp.int32) % N])
    dst = jnp.concatenate(
        [edge_index[1], N + (jnp.arange(pad, dtype=jnp.int32) % 1024)])
    batch2d = batch_node_id.reshape(1, N)

    agg_x = _sc_agg(x, src, dst)
    h1, pool1 = _tc_layer(x, agg_x, batch2d,
                          W1_0, b1_0, gamma_0, beta_0, W2_0, b2_0, True)
    agg_h1 = _sc_agg(h1, src, dst)
    (pool2,) = _tc_layer(h1, agg_h1, batch2d,
                         W1_1, b1_1, gamma_1, beta_1, W2_1, b2_1, False)
    return jnp.concatenate([pool1, pool2], axis=1)


# speedup vs baseline: 3.9906x; 1.1377x over previous
import functools

import jax
import jax.numpy as jnp
from jax import lax
from jax.experimental import pallas as pl
from jax.experimental.pallas import tpu as pltpu
from jax.experimental.pallas import tpu_sc as plsc

N = 10000
E = 320000
D = 128
G = 8

NC = 2
NS = 16
NW = NC * NS
CH = 128
NCHUNK = 84
PADE = NW * NCHUNK * CH
ZR = 1024
NB = 3
NJ = 6
NOUTER = NCHUNK // NJ
RPT = 624
TAIL = N - NS * RPT

_mesh = plsc.VectorSubcoreMesh(core_axis_name="c", subcore_axis_name="s")


@functools.partial(
    pl.kernel,
    out_type=jax.ShapeDtypeStruct((NC, N, D), jnp.float32),
    mesh=_mesh,
    scratch_types=[
        [pltpu.VMEM((CH,), jnp.int32) for _ in range(NJ)],
        [pltpu.VMEM((CH,), jnp.int32) for _ in range(NJ)],
        [pltpu.VMEM((CH, D), jnp.float32) for _ in range(NB)],
        pltpu.VMEM_SHARED((N, D), jnp.float32),
        pltpu.SemaphoreType.DMA((NB,)),
        pltpu.SemaphoreType.DMA((NB,)),
        pltpu.SemaphoreType.DMA((NJ,)),
        pltpu.SemaphoreType.DMA((NJ,)),
    ],
)
def _sc_agg(h_hbm, src_hbm, dst_hbm, out_hbm,
            si, di, rows, acc_sh, sem_g, sem_s, sem_si, sem_di):
    c = lax.axis_index("c")
    s = lax.axis_index("s")
    wid = c * NS + s

    zv = jnp.zeros((16,), jnp.float32)

    def _zrow(i, _):
        def _zcol(j, _):
            rows[0][i, pl.ds(j * 16, 16)] = zv
            return 0
        return lax.fori_loop(0, D // 16, _zcol, 0)

    lax.fori_loop(0, CH, _zrow, 0)

    def _zcp(k, _):
        pltpu.sync_copy(rows[0], acc_sh.at[pl.ds(s * RPT + k * CH, CH)])
        return 0

    lax.fori_loop(0, RPT // CH, _zcp, 0)
    pltpu.sync_copy(rows[0].at[pl.ds(0, RPT % CH)],
                    acc_sh.at[pl.ds(s * RPT + (RPT // CH) * CH, RPT % CH)])

    @pl.when(s == NS - 1)
    def _ztail():
        pltpu.sync_copy(rows[0].at[pl.ds(0, TAIL)],
                        acc_sh.at[pl.ds(NS * RPT, TAIL)])

    plsc.subcore_barrier()

    base = wid * NCHUNK * CH

    def _issue_idx(off, j):
        pltpu.async_copy(src_hbm.at[pl.ds(off, CH)], si[j], sem_si.at[j])
        pltpu.async_copy(dst_hbm.at[pl.ds(off, CH)], di[j], sem_di.at[j])

    def _wait_idx(off, j):
        pltpu.make_async_copy(
            src_hbm.at[pl.ds(off, CH)], si[j], sem_si.at[j]).wait()
        pltpu.make_async_copy(
            dst_hbm.at[pl.ds(off, CH)], di[j], sem_di.at[j]).wait()

    def _issue_gather(j, b):
        pltpu.async_copy(h_hbm.at[si[j]], rows[b], sem_g.at[b])

    def _wait_gather(b):
        pltpu.make_async_copy(h_hbm.at[si[0]], rows[b], sem_g.at[b]).wait()

    def _drain_scatter(b, j):
        pltpu.make_async_copy(rows[b], acc_sh.at[di[j]], sem_s.at[b]).wait()

    for j in range(4):
        _issue_idx(base + j * CH, j)
    for i in range(2):
        _wait_idx(base + i * CH, i)
        _issue_gather(i, i)

    def _outer(g, _):
        for u in range(NJ):
            i_off = base + (NJ * g + u) * CH
            b = u % NB
            j = u

            _wait_gather(b)
            pltpu.async_copy(rows[b], acc_sh.at[di[j]], sem_s.at[b], add=True)

            def _dr(b=b, j=j):
                _drain_scatter((b + NB - 1) % NB, (j + NJ - 1) % NJ)

            if u == 0:
                pl.when(g > 0)(_dr)
            else:
                _dr()

            def _next_gather(i_off=i_off, j=j, b=b):
                _wait_idx(i_off + 2 * CH, (j + 2) % NJ)
                _issue_gather((j + 2) % NJ, (b + 2) % NB)

            if u < 4:
                _next_gather()
            else:
                pl.when(g < NOUTER - 1)(_next_gather)

            def _pf_idx(i_off=i_off, j=j):
                _issue_idx(i_off + 4 * CH, (j + 4) % NJ)

            if u < 2:
                _pf_idx()
            else:
                pl.when(g < NOUTER - 1)(_pf_idx)
        return 0

    lax.fori_loop(0, NOUTER, _outer, 0)

    _drain_scatter((NCHUNK - 1) % NB, (NCHUNK - 1) % NJ)

    plsc.subcore_barrier()

    pltpu.sync_copy(acc_sh.at[pl.ds(s * RPT, RPT)],
                    out_hbm.at[c, pl.ds(s * RPT, RPT)])

    @pl.when(s == NS - 1)
    def _wtail():
        pltpu.sync_copy(acc_sh.at[pl.ds(NS * RPT, TAIL)],
                        out_hbm.at[c, pl.ds(NS * RPT, TAIL)])


def _mlp_pool_body(emit_h, h_ref, agg_ref, batch_ref,
                   W1_ref, b1_ref, g_ref, be_ref, W2_ref, b2_ref, *outs):
    z = h_ref[...] + agg_ref[0] + agg_ref[1]
    z = jnp.dot(z, W1_ref[...], preferred_element_type=jnp.float32) + b1_ref[...]
    mean = jnp.mean(z, axis=0, keepdims=True)
    var = jnp.mean(z * z, axis=0, keepdims=True) - mean * mean
    z = (z - mean) * (g_ref[...] * lax.rsqrt(var + 1e-5)) + be_ref[...]
    z = jnp.maximum(z, 0.0)
    z = jnp.dot(z, W2_ref[...], preferred_element_type=jnp.float32) + b2_ref[...]
    h = jnp.maximum(z, 0.0)
    onehot = (batch_ref[...] ==
              lax.broadcasted_iota(jnp.int32, (G, N), 0)).astype(jnp.float32)
    pool = jnp.dot(onehot, h, preferred_element_type=jnp.float32)
    if emit_h:
        outs[0][...] = h
        outs[1][...] = pool
    else:
        outs[0][...] = pool


def _tc_layer(h, agg2, batch2d, W1, b1, gamma, beta, W2, b2, emit_h):
    if emit_h:
        out_shape = (jax.ShapeDtypeStruct((N, D), jnp.float32),
                     jax.ShapeDtypeStruct((G, D), jnp.float32))
    else:
        out_shape = (jax.ShapeDtypeStruct((G, D), jnp.float32),)
    return pl.pallas_call(
        functools.partial(_mlp_pool_body, emit_h),
        out_shape=out_shape,
    )(h, agg2, batch2d,
      W1, b1.reshape(1, D), gamma.reshape(1, D), beta.reshape(1, D),
      W2, b2.reshape(1, D))


def kernel(x, edge_index, batch_node_id,
           W1_0, b1_0, gamma_0, beta_0, W2_0, b2_0,
           W1_1, b1_1, gamma_1, beta_1, W2_1, b2_1):
    pad = PADE - E
    src = jnp.concatenate(
        [edge_index[0], N + (jnp.arange(pad, dtype=jnp.int32) % ZR)])
    dst = jnp.concatenate(
        [edge_index[1], jnp.arange(pad, dtype=jnp.int32) % N])
    batch2d = batch_node_id.reshape(1, N)
    zpad = jnp.zeros((ZR, D), jnp.float32)

    xz = jnp.concatenate([x, zpad])
    agg_x = _sc_agg(xz, src, dst)
    h1, pool1 = _tc_layer(x, agg_x, batch2d,
                          W1_0, b1_0, gamma_0, beta_0, W2_0, b2_0, True)
    h1z = jnp.concatenate([h1, zpad])
    agg_h1 = _sc_agg(h1z, src, dst)
    (pool2,) = _tc_layer(h1, agg_h1, batch2d,
                         W1_1, b1_1, gamma_1, beta_1, W2_1, b2_1, False)
    return jnp.concatenate([pool1, pool2], axis=1)


# speedup vs baseline: 4.1899x; 1.0499x over previous
import functools

import jax
import jax.numpy as jnp
from jax import lax
from jax.experimental import pallas as pl
from jax.experimental.pallas import tpu as pltpu
from jax.experimental.pallas import tpu_sc as plsc

N = 10000
E = 320000
D = 128
G = 8

NC = 2
NS = 16
NW = NC * NS
CH = 112
NCHUNK = 90
PADE = NW * NCHUNK * CH
ZR = 1024
NB = 3
NJ = 6
NOUTER = NCHUNK // NJ
RPT = 624
TAIL = N - NS * RPT

_mesh = plsc.VectorSubcoreMesh(core_axis_name="c", subcore_axis_name="s")


@functools.partial(
    pl.kernel,
    out_type=jax.ShapeDtypeStruct((NC, N, D), jnp.float32),
    mesh=_mesh,
    scratch_types=[
        [pltpu.VMEM((CH,), jnp.int32) for _ in range(NJ)],
        [pltpu.VMEM((CH,), jnp.int32) for _ in range(NJ)],
        [pltpu.VMEM((CH, D), jnp.float32) for _ in range(NB)],
        pltpu.VMEM_SHARED((N, D), jnp.float32),
        pltpu.SemaphoreType.DMA((NB,)),
        pltpu.SemaphoreType.DMA((NB,)),
        pltpu.SemaphoreType.DMA((NJ,)),
        pltpu.SemaphoreType.DMA((NJ,)),
    ],
)
def _sc_agg(h_hbm, src_hbm, dst_hbm, out_hbm,
            si, di, rows, acc_sh, sem_g, sem_s, sem_si, sem_di):
    c = lax.axis_index("c")
    s = lax.axis_index("s")
    wid = c * NS + s

    zv = jnp.zeros((16,), jnp.float32)

    def _zrow(i, _):
        def _zcol(j, _):
            rows[0][i, pl.ds(j * 16, 16)] = zv
            return 0
        return lax.fori_loop(0, D // 16, _zcol, 0)

    lax.fori_loop(0, CH, _zrow, 0)

    def _zcp(k, _):
        pltpu.sync_copy(rows[0], acc_sh.at[pl.ds(s * RPT + k * CH, CH)])
        return 0

    lax.fori_loop(0, RPT // CH, _zcp, 0)
    pltpu.sync_copy(rows[0].at[pl.ds(0, RPT % CH)],
                    acc_sh.at[pl.ds(s * RPT + (RPT // CH) * CH, RPT % CH)])

    @pl.when(s == NS - 1)
    def _ztail():
        pltpu.sync_copy(rows[0].at[pl.ds(0, TAIL)],
                        acc_sh.at[pl.ds(NS * RPT, TAIL)])

    plsc.subcore_barrier()

    base = wid * NCHUNK * CH

    def _issue_idx(off, j):
        pltpu.async_copy(src_hbm.at[pl.ds(off, CH)], si[j], sem_si.at[j])
        pltpu.async_copy(dst_hbm.at[pl.ds(off, CH)], di[j], sem_di.at[j])

    def _wait_idx(off, j):
        pltpu.make_async_copy(
            src_hbm.at[pl.ds(off, CH)], si[j], sem_si.at[j]).wait()
        pltpu.make_async_copy(
            dst_hbm.at[pl.ds(off, CH)], di[j], sem_di.at[j]).wait()

    def _issue_gather(j, b):
        pltpu.async_copy(h_hbm.at[si[j]], rows[b], sem_g.at[b])

    def _wait_gather(b):
        pltpu.make_async_copy(h_hbm.at[si[0]], rows[b], sem_g.at[b]).wait()

    def _drain_scatter(b, j):
        pltpu.make_async_copy(rows[b], acc_sh.at[di[j]], sem_s.at[b]).wait()

    for j in range(4):
        _issue_idx(base + j * CH, j)
    for i in range(2):
        _wait_idx(base + i * CH, i)
        _issue_gather(i, i)

    def _outer(g, _):
        for u in range(NJ):
            i_off = base + (NJ * g + u) * CH
            b = u % NB
            j = u

            _wait_gather(b)
            pltpu.async_copy(rows[b], acc_sh.at[di[j]], sem_s.at[b], add=True)

            def _dr(b=b, j=j):
                _drain_scatter((b + NB - 1) % NB, (j + NJ - 1) % NJ)

            if u == 0:
                pl.when(g > 0)(_dr)
            else:
                _dr()

            def _next_gather(i_off=i_off, j=j, b=b):
                _wait_idx(i_off + 2 * CH, (j + 2) % NJ)
                _issue_gather((j + 2) % NJ, (b + 2) % NB)

            if u < 4:
                _next_gather()
            else:
                pl.when(g < NOUTER - 1)(_next_gather)

            def _pf_idx(i_off=i_off, j=j):
                _issue_idx(i_off + 4 * CH, (j + 4) % NJ)

            if u < 2:
                _pf_idx()
            else:
                pl.when(g < NOUTER - 1)(_pf_idx)
        return 0

    lax.fori_loop(0, NOUTER, _outer, 0)

    _drain_scatter((NCHUNK - 1) % NB, (NCHUNK - 1) % NJ)

    plsc.subcore_barrier()

    pltpu.sync_copy(acc_sh.at[pl.ds(s * RPT, RPT)],
                    out_hbm.at[c, pl.ds(s * RPT, RPT)])

    @pl.when(s == NS - 1)
    def _wtail():
        pltpu.sync_copy(acc_sh.at[pl.ds(NS * RPT, TAIL)],
                        out_hbm.at[c, pl.ds(NS * RPT, TAIL)])


def _mlp_pool_body(emit_h, h_ref, agg_ref, batch_ref,
                   W1_ref, b1_ref, g_ref, be_ref, W2_ref, b2_ref, *outs):
    z = h_ref[...] + agg_ref[0] + agg_ref[1]
    z = jnp.dot(z, W1_ref[...], preferred_element_type=jnp.float32) + b1_ref[...]
    mean = jnp.mean(z, axis=0, keepdims=True)
    var = jnp.mean(z * z, axis=0, keepdims=True) - mean * mean
    z = (z - mean) * (g_ref[...] * lax.rsqrt(var + 1e-5)) + be_ref[...]
    z = jnp.maximum(z, 0.0)
    z = jnp.dot(z, W2_ref[...], preferred_element_type=jnp.float32) + b2_ref[...]
    h = jnp.maximum(z, 0.0)
    onehot = (batch_ref[...] ==
              lax.broadcasted_iota(jnp.int32, (G, N), 0)).astype(jnp.float32)
    pool = jnp.dot(onehot, h, preferred_element_type=jnp.float32)
    if emit_h:
        outs[0][...] = h
        outs[1][...] = pool
    else:
        outs[0][...] = pool


def _tc_layer(h, agg2, batch2d, W1, b1, gamma, beta, W2, b2, emit_h):
    if emit_h:
        out_shape = (jax.ShapeDtypeStruct((N, D), jnp.float32),
                     jax.ShapeDtypeStruct((G, D), jnp.float32))
    else:
        out_shape = (jax.ShapeDtypeStruct((G, D), jnp.float32),)
    return pl.pallas_call(
        functools.partial(_mlp_pool_body, emit_h),
        out_shape=out_shape,
    )(h, agg2, batch2d,
      W1, b1.reshape(1, D), gamma.reshape(1, D), beta.reshape(1, D),
      W2, b2.reshape(1, D))


def kernel(x, edge_index, batch_node_id,
           W1_0, b1_0, gamma_0, beta_0, W2_0, b2_0,
           W1_1, b1_1, gamma_1, beta_1, W2_1, b2_1):
    pad = PADE - E
    src = jnp.concatenate(
        [edge_index[0], N + (jnp.arange(pad, dtype=jnp.int32) % ZR)])
    dst = jnp.concatenate(
        [edge_index[1], jnp.arange(pad, dtype=jnp.int32) % N])
    batch2d = batch_node_id.reshape(1, N)
    zpad = jnp.zeros((ZR, D), jnp.float32)

    xz = jnp.concatenate([x, zpad])
    agg_x = _sc_agg(xz, src, dst)
    h1, pool1 = _tc_layer(x, agg_x, batch2d,
                          W1_0, b1_0, gamma_0, beta_0, W2_0, b2_0, True)
    h1z = jnp.concatenate([h1, zpad])
    agg_h1 = _sc_agg(h1z, src, dst)
    (pool2,) = _tc_layer(h1, agg_h1, batch2d,
                         W1_1, b1_1, gamma_1, beta_1, W2_1, b2_1, False)
    return jnp.concatenate([pool1, pool2], axis=1)


# speedup vs baseline: 4.3285x; 1.0331x over previous
import functools

import jax
import jax.numpy as jnp
from jax import lax
from jax.experimental import pallas as pl
from jax.experimental.pallas import tpu as pltpu
from jax.experimental.pallas import tpu_sc as plsc

N = 10000
E = 320000
D = 128
G = 8

NC = 2
NS = 16
NW = NC * NS
CH = 112
NCHUNK = 90
PADE = NW * NCHUNK * CH
NB = 3
NJ = 6
NOUTER = NCHUNK // NJ
RPT = 624
TAIL = N - NS * RPT

_mesh = plsc.VectorSubcoreMesh(core_axis_name="c", subcore_axis_name="s")


@functools.partial(
    pl.kernel,
    out_type=jax.ShapeDtypeStruct((NC, N, D), jnp.float32),
    mesh=_mesh,
    scratch_types=[
        [pltpu.VMEM((CH,), jnp.int32) for _ in range(NJ)],
        [pltpu.VMEM((CH,), jnp.int32) for _ in range(NJ)],
        [pltpu.VMEM((CH, D), jnp.float32) for _ in range(NB)],
        pltpu.VMEM_SHARED((N + 512, D), jnp.float32),
        pltpu.SemaphoreType.DMA((NB,)),
        pltpu.SemaphoreType.DMA((NB,)),
        pltpu.SemaphoreType.DMA((NJ,)),
        pltpu.SemaphoreType.DMA((NJ,)),
    ],
)
def _sc_agg(h_hbm, src_hbm, dst_hbm, out_hbm,
            si, di, rows, acc_sh, sem_g, sem_s, sem_si, sem_di):
    c = lax.axis_index("c")
    s = lax.axis_index("s")
    wid = c * NS + s

    zv = jnp.zeros((16,), jnp.float32)

    def _zrow(i, _):
        def _zcol(j, _):
            rows[0][i, pl.ds(j * 16, 16)] = zv
            return 0
        return lax.fori_loop(0, D // 16, _zcol, 0)

    lax.fori_loop(0, CH, _zrow, 0)

    def _zcp(k, _):
        pltpu.sync_copy(rows[0], acc_sh.at[pl.ds(s * RPT + k * CH, CH)])
        return 0

    lax.fori_loop(0, RPT // CH, _zcp, 0)
    pltpu.sync_copy(rows[0].at[pl.ds(0, RPT % CH)],
                    acc_sh.at[pl.ds(s * RPT + (RPT // CH) * CH, RPT % CH)])

    @pl.when(s == NS - 1)
    def _ztail():
        pltpu.sync_copy(rows[0].at[pl.ds(0, TAIL)],
                        acc_sh.at[pl.ds(NS * RPT, TAIL)])

    plsc.subcore_barrier()

    base = wid * NCHUNK * CH

    def _issue_idx(off, j):
        pltpu.async_copy(src_hbm.at[pl.ds(off, CH)], si[j], sem_si.at[j])
        pltpu.async_copy(dst_hbm.at[pl.ds(off, CH)], di[j], sem_di.at[j])

    def _wait_idx(off, j):
        pltpu.make_async_copy(
            src_hbm.at[pl.ds(off, CH)], si[j], sem_si.at[j]).wait()
        pltpu.make_async_copy(
            dst_hbm.at[pl.ds(off, CH)], di[j], sem_di.at[j]).wait()

    def _issue_gather(j, b):
        pltpu.async_copy(h_hbm.at[si[j]], rows[b], sem_g.at[b])

    def _wait_gather(b):
        pltpu.make_async_copy(h_hbm.at[si[0]], rows[b], sem_g.at[b]).wait()

    def _drain_scatter(b, j):
        pltpu.make_async_copy(rows[b], acc_sh.at[di[j]], sem_s.at[b]).wait()

    for j in range(4):
        _issue_idx(base + j * CH, j)
    for i in range(2):
        _wait_idx(base + i * CH, i)
        _issue_gather(i, i)

    def _outer(g, _):
        for u in range(NJ):
            i_off = base + (NJ * g + u) * CH
            b = u % NB
            j = u

            _wait_gather(b)
            pltpu.async_copy(rows[b], acc_sh.at[di[j]], sem_s.at[b], add=True)

            def _dr(b=b, j=j):
                _drain_scatter((b + NB - 1) % NB, (j + NJ - 1) % NJ)

            if u == 0:
                pl.when(g > 0)(_dr)
            else:
                _dr()

            def _next_gather(i_off=i_off, j=j, b=b):
                _wait_idx(i_off + 2 * CH, (j + 2) % NJ)
                _issue_gather((j + 2) % NJ, (b + 2) % NB)

            if u < 4:
                _next_gather()
            else:
                pl.when(g < NOUTER - 1)(_next_gather)

            def _pf_idx(i_off=i_off, j=j):
                _issue_idx(i_off + 4 * CH, (j + 4) % NJ)

            if u < 2:
                _pf_idx()
            else:
                pl.when(g < NOUTER - 1)(_pf_idx)
        return 0

    lax.fori_loop(0, NOUTER, _outer, 0)

    _drain_scatter((NCHUNK - 1) % NB, (NCHUNK - 1) % NJ)

    plsc.subcore_barrier()

    pltpu.sync_copy(acc_sh.at[pl.ds(s * RPT, RPT)],
                    out_hbm.at[c, pl.ds(s * RPT, RPT)])

    @pl.when(s == NS - 1)
    def _wtail():
        pltpu.sync_copy(acc_sh.at[pl.ds(NS * RPT, TAIL)],
                        out_hbm.at[c, pl.ds(NS * RPT, TAIL)])


def _mlp_pool_body(emit_h, h_ref, agg_ref, batch_ref,
                   W1_ref, b1_ref, g_ref, be_ref, W2_ref, b2_ref, *outs):
    z = h_ref[...] + agg_ref[0] + agg_ref[1]
    z = jnp.dot(z, W1_ref[...], preferred_element_type=jnp.float32) + b1_ref[...]
    mean = jnp.mean(z, axis=0, keepdims=True)
    var = jnp.mean(z * z, axis=0, keepdims=True) - mean * mean
    z = (z - mean) * (g_ref[...] * lax.rsqrt(var + 1e-5)) + be_ref[...]
    z = jnp.maximum(z, 0.0)
    z = jnp.dot(z, W2_ref[...], preferred_element_type=jnp.float32) + b2_ref[...]
    h = jnp.maximum(z, 0.0)
    onehot = (batch_ref[...] ==
              lax.broadcasted_iota(jnp.int32, (G, N), 0)).astype(jnp.float32)
    pool = jnp.dot(onehot, h, preferred_element_type=jnp.float32)
    if emit_h:
        outs[0][...] = h
        outs[1][...] = pool
    else:
        outs[0][...] = pool


def _tc_layer(h, agg2, batch2d, W1, b1, gamma, beta, W2, b2, emit_h):
    if emit_h:
        out_shape = (jax.ShapeDtypeStruct((N, D), jnp.float32),
                     jax.ShapeDtypeStruct((G, D), jnp.float32))
    else:
        out_shape = (jax.ShapeDtypeStruct((G, D), jnp.float32),)
    return pl.pallas_call(
        functools.partial(_mlp_pool_body, emit_h),
        out_shape=out_shape,
    )(h, agg2, batch2d,
      W1, b1.reshape(1, D), gamma.reshape(1, D), beta.reshape(1, D),
      W2, b2.reshape(1, D))


def kernel(x, edge_index, batch_node_id,
           W1_0, b1_0, gamma_0, beta_0, W2_0, b2_0,
           W1_1, b1_1, gamma_1, beta_1, W2_1, b2_1):
    pad = PADE - E
    src = jnp.concatenate(
        [edge_index[0], jnp.arange(pad, dtype=jnp.int32) % N])
    dst = jnp.concatenate(
        [edge_index[1], N + (jnp.arange(pad, dtype=jnp.int32) % 512)])
    batch2d = batch_node_id.reshape(1, N)

    agg_x = _sc_agg(x, src, dst)
    h1, pool1 = _tc_layer(x, agg_x, batch2d,
                          W1_0, b1_0, gamma_0, beta_0, W2_0, b2_0, True)
    agg_h1 = _sc_agg(h1, src, dst)
    (pool2,) = _tc_layer(h1, agg_h1, batch2d,
                         W1_1, b1_1, gamma_1, beta_1, W2_1, b2_1, False)
    return jnp.concatenate([pool1, pool2], axis=1)


# speedup vs baseline: 4.3484x; 1.0046x over previous
import functools

import jax
import jax.numpy as jnp
from jax import lax
from jax.experimental import pallas as pl
from jax.experimental.pallas import tpu as pltpu
from jax.experimental.pallas import tpu_sc as plsc

N = 10000
E = 320000
D = 128
G = 8

NC = 2
NS = 16
NW = NC * NS
CH = 112
NCHUNK = 90
PADE = NW * NCHUNK * CH
NB = 3
NJ = 6
NOUTER = NCHUNK // NJ
RPT = 624
TAIL = N - NS * RPT

_mesh = plsc.VectorSubcoreMesh(core_axis_name="c", subcore_axis_name="s")


@functools.partial(
    pl.kernel,
    out_type=jax.ShapeDtypeStruct((NC, N, D), jnp.float32),
    mesh=_mesh,
    scratch_types=[
        [pltpu.VMEM((CH,), jnp.int32) for _ in range(NJ)],
        [pltpu.VMEM((CH,), jnp.int32) for _ in range(NJ)],
        [pltpu.VMEM((CH, D), jnp.float32) for _ in range(NB)],
        pltpu.VMEM_SHARED((N + 512, D), jnp.float32),
        pltpu.SemaphoreType.DMA((NB,)),
        pltpu.SemaphoreType.DMA((NB,)),
        pltpu.SemaphoreType.DMA((NJ,)),
        pltpu.SemaphoreType.DMA((NJ,)),
    ],
)
def _sc_agg(h_hbm, src_hbm, dst_hbm, out_hbm,
            si, di, rows, acc_sh, sem_g, sem_s, sem_si, sem_di):
    c = lax.axis_index("c")
    s = lax.axis_index("s")
    wid = c * NS + s

    base = wid * NCHUNK * CH

    def _issue_idx(off, j):
        pltpu.async_copy(src_hbm.at[pl.ds(off, CH)], si[j], sem_si.at[j])
        pltpu.async_copy(dst_hbm.at[pl.ds(off, CH)], di[j], sem_di.at[j])

    def _wait_idx(off, j):
        pltpu.make_async_copy(
            src_hbm.at[pl.ds(off, CH)], si[j], sem_si.at[j]).wait()
        pltpu.make_async_copy(
            dst_hbm.at[pl.ds(off, CH)], di[j], sem_di.at[j]).wait()

    def _issue_gather(j, b):
        pltpu.async_copy(h_hbm.at[si[j]], rows[b], sem_g.at[b])

    def _wait_gather(b):
        pltpu.make_async_copy(h_hbm.at[si[0]], rows[b], sem_g.at[b]).wait()

    def _drain_scatter(b, j):
        pltpu.make_async_copy(rows[b], acc_sh.at[di[j]], sem_s.at[b]).wait()

    for j in range(4):
        _issue_idx(base + j * CH, j)
    for i in range(2):
        _wait_idx(base + i * CH, i)
        _issue_gather(i, i)

    zv = jnp.zeros((16,), jnp.float32)

    def _zrow(i, _):
        def _zcol(j, _):
            rows[2][i, pl.ds(j * 16, 16)] = zv
            return 0
        return lax.fori_loop(0, D // 16, _zcol, 0)

    lax.fori_loop(0, CH, _zrow, 0)

    def _zcp(k, _):
        pltpu.sync_copy(rows[2], acc_sh.at[pl.ds(s * RPT + k * CH, CH)])
        return 0

    lax.fori_loop(0, RPT // CH, _zcp, 0)
    pltpu.sync_copy(rows[2].at[pl.ds(0, RPT % CH)],
                    acc_sh.at[pl.ds(s * RPT + (RPT // CH) * CH, RPT % CH)])

    @pl.when(s == NS - 1)
    def _ztail():
        pltpu.sync_copy(rows[2].at[pl.ds(0, TAIL)],
                        acc_sh.at[pl.ds(NS * RPT, TAIL)])

    plsc.subcore_barrier()

    def _outer(g, _):
        for u in range(NJ):
            i_off = base + (NJ * g + u) * CH
            b = u % NB
            j = u

            _wait_gather(b)
            pltpu.async_copy(rows[b], acc_sh.at[di[j]], sem_s.at[b], add=True)

            def _dr(b=b, j=j):
                _drain_scatter((b + NB - 1) % NB, (j + NJ - 1) % NJ)

            if u == 0:
                pl.when(g > 0)(_dr)
            else:
                _dr()

            def _next_gather(i_off=i_off, j=j, b=b):
                _wait_idx(i_off + 2 * CH, (j + 2) % NJ)
                _issue_gather((j + 2) % NJ, (b + 2) % NB)

            if u < 4:
                _next_gather()
            else:
                pl.when(g < NOUTER - 1)(_next_gather)

            def _pf_idx(i_off=i_off, j=j):
                _issue_idx(i_off + 4 * CH, (j + 4) % NJ)

            if u < 2:
                _pf_idx()
            else:
                pl.when(g < NOUTER - 1)(_pf_idx)
        return 0

    lax.fori_loop(0, NOUTER, _outer, 0)

    _drain_scatter((NCHUNK - 1) % NB, (NCHUNK - 1) % NJ)

    plsc.subcore_barrier()

    pltpu.sync_copy(acc_sh.at[pl.ds(s * RPT, RPT)],
                    out_hbm.at[c, pl.ds(s * RPT, RPT)])

    @pl.when(s == NS - 1)
    def _wtail():
        pltpu.sync_copy(acc_sh.at[pl.ds(NS * RPT, TAIL)],
                        out_hbm.at[c, pl.ds(NS * RPT, TAIL)])


def _mlp_pool_body(emit_h, h_ref, agg_ref, batch_ref,
                   W1_ref, b1_ref, g_ref, be_ref, W2_ref, b2_ref, *outs):
    z = h_ref[...] + agg_ref[0] + agg_ref[1]
    z = jnp.dot(z, W1_ref[...], preferred_element_type=jnp.float32) + b1_ref[...]
    mean = jnp.mean(z, axis=0, keepdims=True)
    var = jnp.mean(z * z, axis=0, keepdims=True) - mean * mean
    z = (z - mean) * (g_ref[...] * lax.rsqrt(var + 1e-5)) + be_ref[...]
    z = jnp.maximum(z, 0.0)
    z = jnp.dot(z, W2_ref[...], preferred_element_type=jnp.float32) + b2_ref[...]
    h = jnp.maximum(z, 0.0)
    onehot = (batch_ref[...] ==
              lax.broadcasted_iota(jnp.int32, (G, N), 0)).astype(jnp.float32)
    pool = jnp.dot(onehot, h, preferred_element_type=jnp.float32)
    if emit_h:
        outs[0][...] = h
        outs[1][...] = pool
    else:
        outs[0][...] = pool


def _tc_layer(h, agg2, batch2d, W1, b1, gamma, beta, W2, b2, emit_h):
    if emit_h:
        out_shape = (jax.ShapeDtypeStruct((N, D), jnp.float32),
                     jax.ShapeDtypeStruct((G, D), jnp.float32))
    else:
        out_shape = (jax.ShapeDtypeStruct((G, D), jnp.float32),)
    return pl.pallas_call(
        functools.partial(_mlp_pool_body, emit_h),
        out_shape=out_shape,
    )(h, agg2, batch2d,
      W1, b1.reshape(1, D), gamma.reshape(1, D), beta.reshape(1, D),
      W2, b2.reshape(1, D))


def kernel(x, edge_index, batch_node_id,
           W1_0, b1_0, gamma_0, beta_0, W2_0, b2_0,
           W1_1, b1_1, gamma_1, beta_1, W2_1, b2_1):
    pad = PADE - E
    src = jnp.concatenate(
        [edge_index[0], jnp.arange(pad, dtype=jnp.int32) % N])
    dst = jnp.concatenate(
        [edge_index[1], N + (jnp.arange(pad, dtype=jnp.int32) % 512)])
    batch2d = batch_node_id.reshape(1, N)

    agg_x = _sc_agg(x, src, dst)
    h1, pool1 = _tc_layer(x, agg_x, batch2d,
                          W1_0, b1_0, gamma_0, beta_0, W2_0, b2_0, True)
    agg_h1 = _sc_agg(h1, src, dst)
    (pool2,) = _tc_layer(h1, agg_h1, batch2d,
                         W1_1, b1_1, gamma_1, beta_1, W2_1, b2_1, False)
    return jnp.concatenate([pool1, pool2], axis=1)
